# baseline (speedup 1.0000x reference)
import jax
import jax.numpy as jnp
from jax import lax
from jax.experimental import pallas as pl
from jax.experimental.pallas import tpu as pltpu

MEL = 4096
NBUF = 3
ACH = 4
OCH = 2


def _layer_norm(h, g, b):
    mu = jnp.mean(h, axis=1, keepdims=True)
    var = jnp.mean((h - mu) ** 2, axis=1, keepdims=True)
    return (h - mu) / jnp.sqrt(var + 1e-5) * g + b


def _align_body(x_ref, t_ref, mml_ref,
                w1p, w1c, w1n, b1, g1, be1,
                w2p, w2c, w2n, b2, g2, be2, lw, lb,
                out_any, al_any, dp_ref,
                cs_ref, a_buf, o_buf, sem_a, sem_o):
    B = t_ref.shape[1]
    L = t_ref.shape[2]
    D = x_ref.shape[2]
    b = pl.program_id(0)
    p = lax.rem(b, NBUF)

    @pl.when(b == 0)
    def _csum():
        dur_all = t_ref[0].astype(jnp.float32)
        tri = (lax.broadcasted_iota(jnp.int32, (L, L), 0)
               <= lax.broadcasted_iota(jnp.int32, (L, L), 1)).astype(jnp.float32)
        cs_ref[...] = jnp.dot(dur_all, tri, preferred_element_type=jnp.float32)

    def _wait(q):
        pltpu.make_async_copy(a_buf.at[q], al_any.at[0], sem_a.at[q]).wait()
        pltpu.make_async_copy(o_buf.at[q], out_any.at[0], sem_o.at[q]).wait()

    @pl.when(b >= NBUF)
    def _drain():
        _wait(p)

    mv = lax.broadcasted_iota(jnp.int32, (MEL, 1), 0)
    mvf = jnp.where(mv < mml_ref[0, 0], mv, -1).astype(jnp.float32)
    cs_b = cs_ref[pl.ds(b, 1), :]
    dur_b = t_ref[0, pl.ds(b, 1), :].astype(jnp.float32)
    csp_b = cs_b - dur_b
    a = ((cs_b > mvf) & (csp_b <= mvf)).astype(jnp.float32)
    a_buf[p] = a
    o_buf[p] = jnp.dot(a.astype(jnp.bfloat16), x_ref[0].astype(jnp.bfloat16),
                       preferred_element_type=jnp.float32)

    am = MEL // ACH
    for k in range(ACH):
        pltpu.make_async_copy(a_buf.at[p, pl.ds(k * am, am)],
                              al_any.at[b, pl.ds(k * am, am), :],
                              sem_a.at[p]).start()
    om = MEL // OCH
    for k in range(OCH):
        pltpu.make_async_copy(o_buf.at[p, pl.ds(k * om, om)],
                              out_any.at[b, pl.ds(k * om, om), :],
                              sem_o.at[p]).start()

    xb = x_ref[0]
    zr = jnp.zeros((1, D), jnp.float32)
    xp = jnp.concatenate([zr, xb[:-1]], axis=0)
    xn = jnp.concatenate([xb[1:], zr], axis=0)
    h = (jnp.dot(xp, w1p[...], preferred_element_type=jnp.float32)
         + jnp.dot(xb, w1c[...], preferred_element_type=jnp.float32)
         + jnp.dot(xn, w1n[...], preferred_element_type=jnp.float32)
         + b1[...])
    h = jax.nn.relu(_layer_norm(h, g1[...], be1[...]))
    hp = jnp.concatenate([zr, h[:-1]], axis=0)
    hn = jnp.concatenate([h[1:], zr], axis=0)
    h2 = (jnp.dot(hp, w2p[...], preferred_element_type=jnp.float32)
          + jnp.dot(h, w2c[...], preferred_element_type=jnp.float32)
          + jnp.dot(hn, w2n[...], preferred_element_type=jnp.float32)
          + b2[...])
    h2 = jax.nn.relu(_layer_norm(h2, g2[...], be2[...]))
    dp = jnp.exp(jnp.sum(h2 * lw[...], axis=1) + lb[0, 0])
    dp_ref[0] = dp.reshape(1, L)

    @pl.when(b == B - 1)
    def _final():
        for q in range(NBUF):
            _wait(q)


def kernel(x, target, mel_max_length,
           conv1_w, conv1_b, ln1_g, ln1_b,
           conv2_w, conv2_b, ln2_g, ln2_b,
           lin_w, lin_b):
    B, L, D = x.shape
    F = conv1_w.shape[0]
    t3 = target.reshape(1, B, L)
    mml = jnp.asarray(mel_max_length, jnp.int32).reshape(1, 1)
    w1p = conv1_w[:, :, 0].T
    w1c = conv1_w[:, :, 1].T
    w1n = conv1_w[:, :, 2].T
    w2p = conv2_w[:, :, 0].T
    w2c = conv2_w[:, :, 1].T
    w2n = conv2_w[:, :, 2].T
    b1 = conv1_b.reshape(1, F)
    g1 = ln1_g.reshape(1, F)
    be1 = ln1_b.reshape(1, F)
    b2 = conv2_b.reshape(1, F)
    g2 = ln2_g.reshape(1, F)
    be2 = ln2_b.reshape(1, F)
    lw = lin_w.reshape(1, F)
    lb = lin_b.reshape(1, 1)

    const = lambda b: (0, 0)
    wspec = lambda shape: pl.BlockSpec(shape, lambda b: (0, 0))
    out, align, dp3 = pl.pallas_call(
        _align_body,
        grid=(B,),
        in_specs=[
            pl.BlockSpec((1, L, D), lambda b: (b, 0, 0)),
            pl.BlockSpec((1, B, L), lambda b: (0, 0, 0)),
            pl.BlockSpec((1, 1), const),
            wspec((D, F)), wspec((D, F)), wspec((D, F)),
            wspec((1, F)), wspec((1, F)), wspec((1, F)),
            wspec((F, F)), wspec((F, F)), wspec((F, F)),
            wspec((1, F)), wspec((1, F)), wspec((1, F)),
            wspec((1, F)), wspec((1, 1)),
        ],
        out_specs=[
            pl.BlockSpec(memory_space=pl.ANY),
            pl.BlockSpec(memory_space=pl.ANY),
            pl.BlockSpec((1, 1, L), lambda b: (b, 0, 0)),
        ],
        out_shape=[
            jax.ShapeDtypeStruct((B, MEL, D), jnp.float32),
            jax.ShapeDtypeStruct((B, MEL, L), jnp.float32),
            jax.ShapeDtypeStruct((B, 1, L), jnp.float32),
        ],
        scratch_shapes=[
            pltpu.VMEM((B, L), jnp.float32),
            pltpu.VMEM((NBUF, MEL, L), jnp.float32),
            pltpu.VMEM((NBUF, MEL, D), jnp.float32),
            pltpu.SemaphoreType.DMA((NBUF,)),
            pltpu.SemaphoreType.DMA((NBUF,)),
        ],
    )(x, t3, mml, w1p, w1c, w1n, b1, g1, be1,
      w2p, w2c, w2n, b2, g2, be2, lw, lb)
    return (out, align, dp3.reshape(B, L))


# speedup vs baseline: 1.0203x; 1.0203x over previous
import jax
import jax.numpy as jnp
from jax import lax
from jax.experimental import pallas as pl
from jax.experimental.pallas import tpu as pltpu

MEL = 4096
NBUF = 2
ACH = 8
OCH = 4


def _layer_norm(h, g, b):
    mu = jnp.mean(h, axis=1, keepdims=True)
    var = jnp.mean((h - mu) ** 2, axis=1, keepdims=True)
    return (h - mu) / jnp.sqrt(var + 1e-5) * g + b


def _align_body(x_ref, t_ref, mml_ref,
                w1p, w1c, w1n, b1, g1, be1,
                w2p, w2c, w2n, b2, g2, be2, lw, lb,
                out_any, al_any, dp_ref,
                cs_ref, a_buf, o_buf, sem_a, sem_o):
    B = t_ref.shape[1]
    L = t_ref.shape[2]
    D = x_ref.shape[2]
    b = pl.program_id(0)
    p = lax.rem(b, NBUF)

    @pl.when(b == 0)
    def _csum():
        dur_all = t_ref[0].astype(jnp.float32)
        tri = (lax.broadcasted_iota(jnp.int32, (L, L), 0)
               <= lax.broadcasted_iota(jnp.int32, (L, L), 1)).astype(jnp.float32)
        cs_ref[...] = jnp.dot(dur_all, tri, preferred_element_type=jnp.float32)

    def _wait(q):
        pltpu.make_async_copy(a_buf.at[q], al_any.at[0], sem_a.at[q]).wait()
        pltpu.make_async_copy(o_buf.at[q], out_any.at[0], sem_o.at[q]).wait()

    @pl.when(b >= NBUF)
    def _drain():
        _wait(p)

    mv = lax.broadcasted_iota(jnp.int32, (MEL, 1), 0)
    mvf = jnp.where(mv < mml_ref[0, 0], mv, -1).astype(jnp.float32)
    cs_b = cs_ref[pl.ds(b, 1), :]
    dur_b = t_ref[0, pl.ds(b, 1), :].astype(jnp.float32)
    csp_b = cs_b - dur_b
    a = ((cs_b > mvf) & (csp_b <= mvf)).astype(jnp.float32)
    a_buf[p] = a
    o_buf[p] = jnp.dot(a.astype(jnp.bfloat16), x_ref[0].astype(jnp.bfloat16),
                       preferred_element_type=jnp.float32)

    am = MEL // ACH
    for k in range(ACH):
        pltpu.make_async_copy(a_buf.at[p, pl.ds(k * am, am)],
                              al_any.at[b, pl.ds(k * am, am), :],
                              sem_a.at[p]).start()
    om = MEL // OCH
    for k in range(OCH):
        pltpu.make_async_copy(o_buf.at[p, pl.ds(k * om, om)],
                              out_any.at[b, pl.ds(k * om, om), :],
                              sem_o.at[p]).start()

    xb = x_ref[0]
    zr = jnp.zeros((1, D), jnp.float32)
    xp = jnp.concatenate([zr, xb[:-1]], axis=0)
    xn = jnp.concatenate([xb[1:], zr], axis=0)
    h = (jnp.dot(xp, w1p[...], preferred_element_type=jnp.float32)
         + jnp.dot(xb, w1c[...], preferred_element_type=jnp.float32)
         + jnp.dot(xn, w1n[...], preferred_element_type=jnp.float32)
         + b1[...])
    h = jax.nn.relu(_layer_norm(h, g1[...], be1[...]))
    hp = jnp.concatenate([zr, h[:-1]], axis=0)
    hn = jnp.concatenate([h[1:], zr], axis=0)
    h2 = (jnp.dot(hp, w2p[...], preferred_element_type=jnp.float32)
          + jnp.dot(h, w2c[...], preferred_element_type=jnp.float32)
          + jnp.dot(hn, w2n[...], preferred_element_type=jnp.float32)
          + b2[...])
    h2 = jax.nn.relu(_layer_norm(h2, g2[...], be2[...]))
    dp = jnp.exp(jnp.sum(h2 * lw[...], axis=1) + lb[0, 0])
    dp_ref[0] = dp.reshape(1, L)

    @pl.when(b == B - 1)
    def _final():
        for q in range(NBUF):
            _wait(q)


def kernel(x, target, mel_max_length,
           conv1_w, conv1_b, ln1_g, ln1_b,
           conv2_w, conv2_b, ln2_g, ln2_b,
           lin_w, lin_b):
    B, L, D = x.shape
    F = conv1_w.shape[0]
    t3 = target.reshape(1, B, L)
    mml = jnp.asarray(mel_max_length, jnp.int32).reshape(1, 1)
    w1p = conv1_w[:, :, 0].T
    w1c = conv1_w[:, :, 1].T
    w1n = conv1_w[:, :, 2].T
    w2p = conv2_w[:, :, 0].T
    w2c = conv2_w[:, :, 1].T
    w2n = conv2_w[:, :, 2].T
    b1 = conv1_b.reshape(1, F)
    g1 = ln1_g.reshape(1, F)
    be1 = ln1_b.reshape(1, F)
    b2 = conv2_b.reshape(1, F)
    g2 = ln2_g.reshape(1, F)
    be2 = ln2_b.reshape(1, F)
    lw = lin_w.reshape(1, F)
    lb = lin_b.reshape(1, 1)

    const = lambda b: (0, 0)
    wspec = lambda shape: pl.BlockSpec(shape, lambda b: (0, 0))
    out, align, dp3 = pl.pallas_call(
        _align_body,
        grid=(B,),
        in_specs=[
            pl.BlockSpec((1, L, D), lambda b: (b, 0, 0)),
            pl.BlockSpec((1, B, L), lambda b: (0, 0, 0)),
            pl.BlockSpec((1, 1), const),
            wspec((D, F)), wspec((D, F)), wspec((D, F)),
            wspec((1, F)), wspec((1, F)), wspec((1, F)),
            wspec((F, F)), wspec((F, F)), wspec((F, F)),
            wspec((1, F)), wspec((1, F)), wspec((1, F)),
            wspec((1, F)), wspec((1, 1)),
        ],
        out_specs=[
            pl.BlockSpec(memory_space=pl.ANY),
            pl.BlockSpec(memory_space=pl.ANY),
            pl.BlockSpec((1, 1, L), lambda b: (b, 0, 0)),
        ],
        out_shape=[
            jax.ShapeDtypeStruct((B, MEL, D), jnp.float32),
            jax.ShapeDtypeStruct((B, MEL, L), jnp.float32),
            jax.ShapeDtypeStruct((B, 1, L), jnp.float32),
        ],
        scratch_shapes=[
            pltpu.VMEM((B, L), jnp.float32),
            pltpu.VMEM((NBUF, MEL, L), jnp.float32),
            pltpu.VMEM((NBUF, MEL, D), jnp.float32),
            pltpu.SemaphoreType.DMA((NBUF,)),
            pltpu.SemaphoreType.DMA((NBUF,)),
        ],
    )(x, t3, mml, w1p, w1c, w1n, b1, g1, be1,
      w2p, w2c, w2n, b2, g2, be2, lw, lb)
    return (out, align, dp3.reshape(B, L))
